# baseline (speedup 1.0000x reference)
import functools

import jax
import jax.numpy as jnp
from jax import lax
from jax.experimental import pallas as pl
from jax.experimental.pallas import tpu as pltpu
from jax.experimental.pallas import tpu_sc as plsc

D_MODEL = 2048
MAX_LEN = 4096
BATCH = 4
SEQ = 4096
LANES = 16

NUM_CORES = 2
NUM_SUBCORES = 16
NW = NUM_CORES * NUM_SUBCORES
TOKENS = BATCH * SEQ
S_PER_W = SEQ // NW
K = 8
SC_CHUNKS = S_PER_W // K
GROUPS = D_MODEL // LANES


def _positional_encoding(max_len, d_model):
    pos = jnp.arange(0, max_len, dtype=jnp.float32)[:, None]
    _2i = jnp.arange(0, d_model, 2, dtype=jnp.float32)
    angle = pos / jnp.power(10000.0, _2i / d_model)
    enc = jnp.zeros((max_len, d_model), dtype=jnp.float32)
    enc = enc.at[:, 0::2].set(jnp.sin(angle))
    enc = enc.at[:, 1::2].set(jnp.cos(angle))
    return enc


@functools.partial(
    pl.kernel,
    mesh=plsc.VectorSubcoreMesh(core_axis_name="c", subcore_axis_name="s"),
    out_type=jax.ShapeDtypeStruct((TOKENS, D_MODEL), jnp.float32),
    scratch_types=[
        pltpu.VMEM((BATCH * S_PER_W,), jnp.int32),
        pltpu.VMEM((K, D_MODEL), jnp.float32),
        pltpu.VMEM((K, D_MODEL), jnp.float32),
        pltpu.VMEM((K, D_MODEL), jnp.float32),
        pltpu.VMEM((K, D_MODEL), jnp.float32),
        pltpu.VMEM((K, D_MODEL), jnp.float32),
        pltpu.VMEM((K, D_MODEL), jnp.float32),
        pltpu.SemaphoreType.DMA,
        pltpu.SemaphoreType.DMA,
        pltpu.SemaphoreType.DMA,
        pltpu.SemaphoreType.DMA,
        pltpu.SemaphoreType.DMA,
        pltpu.SemaphoreType.DMA,
    ],
)
def _emb_sc(x_hbm, table_hbm, pos_hbm, out_hbm,
            idx_v, rows0, rows1, pr0, pr1, ob0, ob1,
            sr0, sr1, sp0, sp1, so0, so1):
    rows = (rows0, rows1)
    pring = (pr0, pr1)
    ob = (ob0, ob1)
    sr = (sr0, sr1)
    sp = (sp0, sp1)
    so = (so0, so1)

    wid = lax.axis_index("s") * NUM_CORES + lax.axis_index("c")
    s0 = wid * S_PER_W
    for b4 in range(BATCH):
        pltpu.sync_copy(x_hbm.at[pl.ds(b4 * SEQ + s0, S_PER_W)],
                        idx_v.at[pl.ds(b4 * S_PER_W, S_PER_W)])

    def issue_rows(sc, b4, slot):
        off = b4 * S_PER_W + sc * K
        pltpu.async_copy(table_hbm.at[idx_v.at[pl.ds(off, K)]],
                         rows[slot], sr[slot])

    def wait_rows(slot):
        pltpu.make_async_copy(table_hbm.at[idx_v.at[pl.ds(0, K)]],
                              rows[slot], sr[slot]).wait()

    def issue_pos(sc, slot):
        pltpu.async_copy(pos_hbm.at[pl.ds(s0 + sc * K, K), :],
                         pring[slot], sp[slot])

    def wait_pos(slot):
        pltpu.make_async_copy(pos_hbm.at[pl.ds(s0, K), :],
                              pring[slot], sp[slot]).wait()

    def wait_out(slot):
        pltpu.make_async_copy(ob[slot], out_hbm.at[pl.ds(0, K), :],
                              so[slot]).wait()

    issue_pos(0, 0)
    issue_pos(1, 1)
    issue_rows(0, 0, 0)
    issue_rows(0, 1, 1)

    def pair_body(sc2, carry):
        for sphase in range(2):
            sc = sc2 * 2 + sphase
            for b4 in range(BATCH):
                slot = b4 % 2
                wait_rows(slot)
                if b4 == 0:
                    wait_pos(sphase)
                if b4 >= 2:
                    wait_out(slot)
                else:
                    @pl.when(sc > 0)
                    def _():
                        wait_out(slot)

                pr_b, rows_b, ob_b = pring[sphase], rows[slot], ob[slot]

                @plsc.parallel_loop(0, K * GROUPS, unroll=8)
                def _(g):
                    r = g // GROUPS
                    sl = pl.ds((g % GROUPS) * LANES, LANES)
                    ob_b[r, sl] = rows_b[r, sl] + pr_b[r, sl]

                pltpu.async_copy(
                    ob[slot],
                    out_hbm.at[pl.ds(b4 * SEQ + s0 + sc * K, K), :],
                    so[slot])

                if b4 < 2:
                    issue_rows(sc, b4 + 2, slot)
                else:
                    @pl.when(sc + 1 < SC_CHUNKS)
                    def _():
                        issue_rows(sc + 1, b4 - 2, slot)
                if b4 == 3:
                    @pl.when(sc + 2 < SC_CHUNKS)
                    def _():
                        issue_pos(sc + 2, sphase)
        return carry

    lax.fori_loop(0, SC_CHUNKS // 2, pair_body, 0)
    wait_out(0)
    wait_out(1)


def kernel(x, emb_weight):
    pos = _positional_encoding(MAX_LEN, D_MODEL)[:SEQ, :]
    x_flat = jnp.reshape(x, (TOKENS,))
    out = _emb_sc(x_flat, emb_weight, pos)
    return jnp.reshape(out, (BATCH, SEQ, D_MODEL))
